# Initial kernel scaffold; baseline (speedup 1.0000x reference)
import jax
import jax.numpy as jnp
from jax.experimental import pallas as pl


def kernel(logits, ids, lengths):
    raise NotImplementedError("write your pallas kernel here")


# speedup vs baseline: 60.1198x; 60.1198x over previous
import functools

import jax
import jax.numpy as jnp
from jax import lax
from jax.experimental import pallas as pl
from jax.experimental.pallas import tpu as pltpu
from jax.experimental.pallas import tpu_sc as plsc

_VOCAB = 32000
_B = 16
_M = 8
_MAXL = 2048
_NROWS = _B * _M
_NW = 32
_RPW = _NROWS // _NW
_LANES = 16


def _sc_row_sums(logits, ids, lengths):
    mesh = plsc.VectorSubcoreMesh(core_axis_name="c", subcore_axis_name="s")

    @functools.partial(
        pl.kernel,
        mesh=mesh,
        compiler_params=pltpu.CompilerParams(needs_layout_passes=False),
        out_type=jax.ShapeDtypeStruct((_NW, _RPW * _LANES), jnp.float32),
        scratch_types=[
            pltpu.VMEM((_VOCAB,), jnp.float32),
            pltpu.VMEM((_RPW, _MAXL), jnp.int32),
            pltpu.VMEM((_NROWS,), jnp.int32),
            pltpu.VMEM((_RPW * _LANES,), jnp.float32),
            pltpu.SemaphoreType.DMA,
        ],
    )
    def body(logits_hbm, ids_hbm, lens_hbm, g_hbm,
             table_v, ids_v, lens_v, gbuf_v, sem):
        cid = lax.axis_index("c")
        sid = lax.axis_index("s")
        wid = sid * 2 + cid
        row0 = wid * _RPW

        c1 = pltpu.async_copy(logits_hbm, table_v, sem)
        c2 = pltpu.async_copy(ids_hbm.at[pl.ds(row0, _RPW)], ids_v, sem)
        c3 = pltpu.async_copy(lens_hbm, lens_v, sem)
        c1.wait()
        c2.wait()
        c3.wait()

        iota = lax.iota(jnp.int32, _LANES)
        for j in range(_RPW):
            len_splat = plsc.load_gather(
                lens_v, [jnp.full((_LANES,), row0 + j, jnp.int32)])
            lenr = jnp.maximum(len_splat[0], 1)
            nsteps = (lenr + _LANES - 1) // _LANES

            def step(t, acc, j=j, lenr=lenr):
                idv = ids_v[j, pl.ds(t * _LANES, _LANES)]
                g = plsc.load_gather(table_v, [idv])
                msk = (t * _LANES + iota) < lenr
                return acc + jnp.where(msk, g, jnp.zeros_like(g))

            acc = lax.fori_loop(0, nsteps, step,
                                jnp.zeros((_LANES,), jnp.float32))
            s = jnp.sum(acc)
            gbuf_v[pl.ds(_LANES * j, _LANES)] = jnp.full(
                (_LANES,), s, jnp.float32)

        pltpu.sync_copy(gbuf_v, g_hbm.at[wid])

    return body(logits, ids, lengths)


def _tc_finalize(logits2d, g, lens_i):
    def body(lg_ref, g_ref, len_ref, attn_ref, logps_ref, loss_ref):
        x = lg_ref[...]
        mx = jnp.max(x)
        lse = mx + jnp.log(jnp.sum(jnp.exp(x - mx)))
        lens = jnp.maximum(len_ref[...], 1).astype(jnp.float32)
        logps = g_ref[...] - lens * lse
        rowmax = jnp.max(logps, axis=1, keepdims=True)
        e = jnp.exp(logps - rowmax)
        attn = e / jnp.sum(e, axis=1, keepdims=True)
        attn_ref[...] = attn
        logps_ref[...] = logps
        loss_ref[...] = jnp.reshape(
            jnp.sum(-logps * attn / lens) / _NROWS, (1, 1))

    return pl.pallas_call(
        body,
        out_shape=(
            jax.ShapeDtypeStruct((_B, _M), jnp.float32),
            jax.ShapeDtypeStruct((_B, _M), jnp.float32),
            jax.ShapeDtypeStruct((1, 1), jnp.float32),
        ),
    )(logits2d, g, lens_i)


@jax.jit
def _impl(logits, ids, lengths):
    g_raw = _sc_row_sums(logits, ids, lengths)
    g = g_raw.reshape(_NROWS, _LANES)[:, 0].reshape(_B, _M)
    logits2d = jnp.pad(logits, (0, 32768 - _VOCAB),
                       constant_values=-1e30).reshape(256, 128)
    lens_i = lengths.reshape(_B, _M)
    attn, logps, loss = _tc_finalize(logits2d, g, lens_i)
    return attn, logps, loss[0, 0]


def kernel(logits, ids, lengths):
    return _impl(logits, ids, lengths)


# speedup vs baseline: 62.2964x; 1.0362x over previous
import functools

import jax
import jax.numpy as jnp
from jax import lax
from jax.experimental import pallas as pl
from jax.experimental.pallas import tpu as pltpu
from jax.experimental.pallas import tpu_sc as plsc

_VOCAB = 32000
_B = 16
_M = 8
_MAXL = 2048
_NROWS = _B * _M
_NW = 32
_RPW = _NROWS // _NW
_LANES = 16


def _sc_row_sums(logits, ids, lengths):
    mesh = plsc.VectorSubcoreMesh(core_axis_name="c", subcore_axis_name="s")

    @functools.partial(
        pl.kernel,
        mesh=mesh,
        compiler_params=pltpu.CompilerParams(needs_layout_passes=False),
        out_type=jax.ShapeDtypeStruct((_NROWS, _LANES), jnp.float32),
        scratch_types=[
            pltpu.VMEM((_VOCAB,), jnp.float32),
            pltpu.VMEM((_RPW, _MAXL), jnp.int32),
            pltpu.VMEM((_NROWS,), jnp.int32),
            pltpu.VMEM((_RPW, _LANES), jnp.float32),
            pltpu.SemaphoreType.DMA,
        ],
    )
    def body(logits_hbm, ids_hbm, lens_hbm, g_hbm,
             table_v, ids_v, lens_v, gbuf_v, sem):
        cid = lax.axis_index("c")
        sid = lax.axis_index("s")
        wid = sid * 2 + cid
        row0 = wid * _RPW

        c1 = pltpu.async_copy(logits_hbm, table_v, sem)
        c2 = pltpu.async_copy(ids_hbm.at[pl.ds(row0, _RPW)], ids_v, sem)
        c3 = pltpu.async_copy(lens_hbm, lens_v, sem)
        c1.wait()
        c2.wait()
        c3.wait()

        iota = lax.iota(jnp.int32, _LANES)
        for j in range(_RPW):
            len_splat = plsc.load_gather(
                lens_v, [jnp.full((_LANES,), row0 + j, jnp.int32)])
            lenr = jnp.maximum(len_splat[0], 1)
            nsteps = (lenr + _LANES - 1) // _LANES

            def step(t, acc, j=j, lenr=lenr):
                idv = ids_v[j, pl.ds(t * _LANES, _LANES)]
                g = plsc.load_gather(table_v, [idv])
                msk = (t * _LANES + iota) < lenr
                return acc + jnp.where(msk, g, jnp.zeros_like(g))

            acc = lax.fori_loop(0, nsteps, step,
                                jnp.zeros((_LANES,), jnp.float32))
            gbuf_v[j] = acc

        pltpu.sync_copy(gbuf_v, g_hbm.at[pl.ds(row0, _RPW)])

    return body(logits, ids, lengths)


def _tc_finalize(logits2d, g, lens_i):
    def body(lg_ref, g_ref, len_ref, attn_ref, logps_ref, loss_ref):
        x = lg_ref[...]
        mx = jnp.max(x)
        lse = mx + jnp.log(jnp.sum(jnp.exp(x - mx)))
        lens = jnp.maximum(len_ref[...], 1).astype(jnp.float32)
        g = jnp.sum(g_ref[...], axis=1).reshape(_B, _M)
        logps = g - lens * lse
        rowmax = jnp.max(logps, axis=1, keepdims=True)
        e = jnp.exp(logps - rowmax)
        attn = e / jnp.sum(e, axis=1, keepdims=True)
        attn_ref[...] = attn
        logps_ref[...] = logps
        loss_ref[...] = jnp.reshape(
            jnp.sum(-logps * attn / lens) / _NROWS, (1, 1))

    return pl.pallas_call(
        body,
        out_shape=(
            jax.ShapeDtypeStruct((_B, _M), jnp.float32),
            jax.ShapeDtypeStruct((_B, _M), jnp.float32),
            jax.ShapeDtypeStruct((1, 1), jnp.float32),
        ),
    )(logits2d, g, lens_i)


@jax.jit
def _impl(logits, ids, lengths):
    g_raw = _sc_row_sums(logits, ids, lengths)
    logits2d = logits.reshape(250, 128)
    lens_i = lengths.reshape(_B, _M)
    attn, logps, loss = _tc_finalize(logits2d, g_raw, lens_i)
    return attn, logps, loss[0, 0]


def kernel(logits, ids, lengths):
    return _impl(logits, ids, lengths)


# speedup vs baseline: 64.8610x; 1.0412x over previous
import functools

import jax
import jax.numpy as jnp
from jax import lax
from jax.experimental import pallas as pl
from jax.experimental.pallas import tpu as pltpu
from jax.experimental.pallas import tpu_sc as plsc

_VOCAB = 32000
_B = 16
_M = 8
_MAXL = 2048
_NROWS = _B * _M
_NW = 32
_RPW = _NROWS // _NW
_LANES = 16


def _sc_row_sums(logits, ids, lengths):
    mesh = plsc.VectorSubcoreMesh(core_axis_name="c", subcore_axis_name="s")

    @functools.partial(
        pl.kernel,
        mesh=mesh,
        compiler_params=pltpu.CompilerParams(needs_layout_passes=False),
        out_type=jax.ShapeDtypeStruct((_NROWS, _LANES), jnp.float32),
        scratch_types=[
            pltpu.VMEM((_VOCAB,), jnp.float32),
            pltpu.VMEM((_RPW, _MAXL), jnp.int32),
            pltpu.VMEM((_NROWS,), jnp.int32),
            pltpu.VMEM((_RPW, _LANES), jnp.float32),
            pltpu.SemaphoreType.DMA,
        ],
    )
    def body(logits_hbm, ids_hbm, lens_hbm, g_hbm,
             table_v, ids_v, lens_v, gbuf_v, sem):
        cid = lax.axis_index("c")
        sid = lax.axis_index("s")
        wid = sid * 2 + cid
        row0 = wid * _RPW

        c1 = pltpu.async_copy(logits_hbm, table_v, sem)
        c2 = pltpu.async_copy(ids_hbm.at[pl.ds(row0, _RPW)], ids_v, sem)
        c3 = pltpu.async_copy(lens_hbm, lens_v, sem)
        c1.wait()
        c2.wait()
        c3.wait()

        iota = lax.iota(jnp.int32, _LANES)
        unroll = 8
        chunk = unroll * _LANES
        for j in range(_RPW):
            len_splat = plsc.load_gather(
                lens_v, [jnp.full((_LANES,), row0 + j, jnp.int32)])
            lenr = jnp.maximum(len_splat[0], 1)
            nchunks = (lenr + chunk - 1) // chunk

            def step(t, acc, j=j, lenr=lenr):
                base = t * chunk
                for u in range(unroll):
                    idv = ids_v[j, pl.ds(base + u * _LANES, _LANES)]
                    g = plsc.load_gather(table_v, [idv])
                    msk = (base + u * _LANES + iota) < lenr
                    acc = acc + jnp.where(msk, g, jnp.zeros_like(g))
                return acc

            acc = lax.fori_loop(0, nchunks, step,
                                jnp.zeros((_LANES,), jnp.float32))
            gbuf_v[j] = acc

        pltpu.sync_copy(gbuf_v, g_hbm.at[pl.ds(row0, _RPW)])

    return body(logits, ids, lengths)


def _tc_finalize(logits2d, g, lens_i):
    def body(lg_ref, g_ref, len_ref, attn_ref, logps_ref, loss_ref):
        x = lg_ref[...]
        mx = jnp.max(x)
        lse = mx + jnp.log(jnp.sum(jnp.exp(x - mx)))
        lens = jnp.maximum(len_ref[...], 1).astype(jnp.float32)
        g = jnp.sum(g_ref[...], axis=1).reshape(_B, _M)
        logps = g - lens * lse
        rowmax = jnp.max(logps, axis=1, keepdims=True)
        e = jnp.exp(logps - rowmax)
        attn = e / jnp.sum(e, axis=1, keepdims=True)
        attn_ref[...] = attn
        logps_ref[...] = logps
        loss_ref[...] = jnp.reshape(
            jnp.sum(-logps * attn / lens) / _NROWS, (1, 1))

    return pl.pallas_call(
        body,
        out_shape=(
            jax.ShapeDtypeStruct((_B, _M), jnp.float32),
            jax.ShapeDtypeStruct((_B, _M), jnp.float32),
            jax.ShapeDtypeStruct((1, 1), jnp.float32),
        ),
    )(logits2d, g, lens_i)


@jax.jit
def _impl(logits, ids, lengths):
    g_raw = _sc_row_sums(logits, ids, lengths)
    logits2d = logits.reshape(250, 128)
    lens_i = lengths.reshape(_B, _M)
    attn, logps, loss = _tc_finalize(logits2d, g_raw, lens_i)
    return attn, logps, loss[0, 0]


def kernel(logits, ids, lengths):
    return _impl(logits, ids, lengths)


# speedup vs baseline: 64.8885x; 1.0004x over previous
import functools

import jax
import jax.numpy as jnp
from jax import lax
from jax.experimental import pallas as pl
from jax.experimental.pallas import tpu as pltpu
from jax.experimental.pallas import tpu_sc as plsc

_VOCAB = 32000
_B = 16
_M = 8
_MAXL = 2048
_NROWS = _B * _M
_NW = 32
_RPW = _NROWS // _NW
_LANES = 16


def _sc_row_sums(logits, ids, lengths):
    mesh = plsc.VectorSubcoreMesh(core_axis_name="c", subcore_axis_name="s")

    @functools.partial(
        pl.kernel,
        mesh=mesh,
        compiler_params=pltpu.CompilerParams(needs_layout_passes=False),
        out_type=jax.ShapeDtypeStruct((_NROWS, _LANES), jnp.float32),
        scratch_types=[
            pltpu.VMEM((_VOCAB,), jnp.float32),
            pltpu.VMEM((_RPW, _MAXL), jnp.int32),
            pltpu.VMEM((_NROWS,), jnp.int32),
            pltpu.VMEM((_RPW, _LANES), jnp.float32),
            pltpu.SemaphoreType.DMA,
        ],
    )
    def body(logits_hbm, ids_hbm, lens_hbm, g_hbm,
             table_v, ids_v, lens_v, gbuf_v, sem):
        cid = lax.axis_index("c")
        sid = lax.axis_index("s")
        wid = sid * 2 + cid
        row0 = wid * _RPW

        nsplit = 4
        seg = _VOCAB // nsplit
        copies = [
            pltpu.async_copy(logits_hbm.at[pl.ds(k * seg, seg)],
                             table_v.at[pl.ds(k * seg, seg)], sem)
            for k in range(nsplit)
        ]
        copies.append(
            pltpu.async_copy(ids_hbm.at[pl.ds(row0, _RPW)], ids_v, sem))
        copies.append(pltpu.async_copy(lens_hbm, lens_v, sem))
        for c in copies:
            c.wait()

        iota = lax.iota(jnp.int32, _LANES)
        unroll = 8
        chunk = unroll * _LANES
        for j in range(_RPW):
            len_splat = plsc.load_gather(
                lens_v, [jnp.full((_LANES,), row0 + j, jnp.int32)])
            lenr = jnp.maximum(len_splat[0], 1)
            nchunks = (lenr + chunk - 1) // chunk

            def step(t, acc, j=j, lenr=lenr):
                base = t * chunk
                for u in range(unroll):
                    idv = ids_v[j, pl.ds(base + u * _LANES, _LANES)]
                    g = plsc.load_gather(table_v, [idv])
                    msk = (base + u * _LANES + iota) < lenr
                    acc = acc + jnp.where(msk, g, jnp.zeros_like(g))
                return acc

            acc = lax.fori_loop(0, nchunks, step,
                                jnp.zeros((_LANES,), jnp.float32))
            gbuf_v[j] = acc

        pltpu.sync_copy(gbuf_v, g_hbm.at[pl.ds(row0, _RPW)])

    return body(logits, ids, lengths)


def _tc_finalize(logits2d, g, lens_i):
    def body(lg_ref, g_ref, len_ref, attn_ref, logps_ref, loss_ref):
        x = lg_ref[...]
        mx = jnp.max(x)
        lse = mx + jnp.log(jnp.sum(jnp.exp(x - mx)))
        lens = jnp.maximum(len_ref[...], 1).astype(jnp.float32)
        g = jnp.sum(g_ref[...], axis=1).reshape(_B, _M)
        logps = g - lens * lse
        rowmax = jnp.max(logps, axis=1, keepdims=True)
        e = jnp.exp(logps - rowmax)
        attn = e / jnp.sum(e, axis=1, keepdims=True)
        attn_ref[...] = attn
        logps_ref[...] = logps
        loss_ref[...] = jnp.reshape(
            jnp.sum(-logps * attn / lens) / _NROWS, (1, 1))

    return pl.pallas_call(
        body,
        out_shape=(
            jax.ShapeDtypeStruct((_B, _M), jnp.float32),
            jax.ShapeDtypeStruct((_B, _M), jnp.float32),
            jax.ShapeDtypeStruct((1, 1), jnp.float32),
        ),
    )(logits2d, g, lens_i)


@jax.jit
def _impl(logits, ids, lengths):
    g_raw = _sc_row_sums(logits, ids, lengths)
    logits2d = logits.reshape(250, 128)
    lens_i = lengths.reshape(_B, _M)
    attn, logps, loss = _tc_finalize(logits2d, g_raw, lens_i)
    return attn, logps, loss[0, 0]


def kernel(logits, ids, lengths):
    return _impl(logits, ids, lengths)


# speedup vs baseline: 72.6446x; 1.1195x over previous
import functools

import jax
import jax.numpy as jnp
from jax import lax
from jax.experimental import pallas as pl
from jax.experimental.pallas import tpu as pltpu
from jax.experimental.pallas import tpu_sc as plsc

_VOCAB = 32000
_B = 16
_M = 8
_MAXL = 2048
_NROWS = _B * _M
_NW = 32
_RPW = _NROWS // _NW
_LANES = 16


def _sc_row_sums(logits, ids, lengths):
    mesh = plsc.VectorSubcoreMesh(core_axis_name="c", subcore_axis_name="s")

    @functools.partial(
        pl.kernel,
        mesh=mesh,
        compiler_params=pltpu.CompilerParams(needs_layout_passes=False),
        out_type=jax.ShapeDtypeStruct((_NROWS, _LANES), jnp.float32),
        scratch_types=[
            pltpu.VMEM((_VOCAB,), jnp.float32),
            pltpu.VMEM((_RPW, _MAXL), jnp.int32),
            pltpu.VMEM((_NROWS,), jnp.int32),
            pltpu.VMEM((_RPW, _LANES), jnp.float32),
            pltpu.VMEM_SHARED((_VOCAB,), jnp.float32),
            pltpu.SemaphoreType.DMA,
        ],
    )
    def body(logits_hbm, ids_hbm, lens_hbm, g_hbm,
             table_v, ids_v, lens_v, gbuf_v, table_sh, sem):
        cid = lax.axis_index("c")
        sid = lax.axis_index("s")
        wid = sid * 2 + cid
        row0 = wid * _RPW

        copies = [
            pltpu.async_copy(ids_hbm.at[pl.ds(row0, _RPW)], ids_v, sem),
            pltpu.async_copy(lens_hbm, lens_v, sem),
        ]

        @pl.when(sid == 0)
        def _stage():
            pltpu.sync_copy(logits_hbm, table_sh)

        plsc.subcore_barrier()
        pltpu.sync_copy(table_sh, table_v)
        for c in copies:
            c.wait()

        iota = lax.iota(jnp.int32, _LANES)
        unroll = 8
        chunk = unroll * _LANES
        for j in range(_RPW):
            len_splat = plsc.load_gather(
                lens_v, [jnp.full((_LANES,), row0 + j, jnp.int32)])
            lenr = jnp.maximum(len_splat[0], 1)
            nchunks = (lenr + chunk - 1) // chunk

            def step(t, acc, j=j, lenr=lenr):
                base = t * chunk
                for u in range(unroll):
                    idv = ids_v[j, pl.ds(base + u * _LANES, _LANES)]
                    g = plsc.load_gather(table_v, [idv])
                    msk = (base + u * _LANES + iota) < lenr
                    acc = acc + jnp.where(msk, g, jnp.zeros_like(g))
                return acc

            acc = lax.fori_loop(0, nchunks, step,
                                jnp.zeros((_LANES,), jnp.float32))
            gbuf_v[j] = acc

        pltpu.sync_copy(gbuf_v, g_hbm.at[pl.ds(row0, _RPW)])

    return body(logits, ids, lengths)


def _tc_finalize(logits2d, g, lens_i):
    def body(lg_ref, g_ref, len_ref, attn_ref, logps_ref, loss_ref):
        x = lg_ref[...]
        mx = jnp.max(x)
        lse = mx + jnp.log(jnp.sum(jnp.exp(x - mx)))
        lens = jnp.maximum(len_ref[...], 1).astype(jnp.float32)
        g = jnp.sum(g_ref[...], axis=1).reshape(_B, _M)
        logps = g - lens * lse
        rowmax = jnp.max(logps, axis=1, keepdims=True)
        e = jnp.exp(logps - rowmax)
        attn = e / jnp.sum(e, axis=1, keepdims=True)
        attn_ref[...] = attn
        logps_ref[...] = logps
        loss_ref[...] = jnp.reshape(
            jnp.sum(-logps * attn / lens) / _NROWS, (1, 1))

    return pl.pallas_call(
        body,
        out_shape=(
            jax.ShapeDtypeStruct((_B, _M), jnp.float32),
            jax.ShapeDtypeStruct((_B, _M), jnp.float32),
            jax.ShapeDtypeStruct((1, 1), jnp.float32),
        ),
    )(logits2d, g, lens_i)


@jax.jit
def _impl(logits, ids, lengths):
    g_raw = _sc_row_sums(logits, ids, lengths)
    logits2d = logits.reshape(250, 128)
    lens_i = lengths.reshape(_B, _M)
    attn, logps, loss = _tc_finalize(logits2d, g_raw, lens_i)
    return attn, logps, loss[0, 0]


def kernel(logits, ids, lengths):
    return _impl(logits, ids, lengths)
